# baseline (speedup 1.0000x reference)
import dataclasses
import functools

import jax
import jax.numpy as jnp
from jax import lax
from jax.experimental import pallas as pl
from jax.experimental.pallas import tpu as pltpu
from jax.experimental.pallas import tpu_sc as plsc

V = 1000
VP = 1024
B = 4096
T = 20
N = B * T
L = 16
NW = 32
BPT = B // NW
CPT = N // NW
K = 16
NC = CPT // K
RING = 80
GS = RING // K
OS = RING // T


def _lse_rows(table):
    def body(t_ref, o_ref):
        t = t_ref[...]
        m = jnp.max(t, axis=1, keepdims=True)
        s = jnp.sum(jnp.exp(t - m), axis=1, keepdims=True)
        o_ref[...] = jnp.log(s) + m

    return pl.pallas_call(
        body,
        out_shape=jax.ShapeDtypeStruct((V, 1), jnp.float32),
    )(table)


def _mean_partials(partials):
    def body(p_ref, o_ref):
        s = jnp.sum(p_ref[...], axis=1, keepdims=True)
        o_ref[...] = jnp.sum(s, axis=0, keepdims=True) * jnp.float32(1.0 / N)

    return pl.pallas_call(
        body,
        out_shape=jax.ShapeDtypeStruct((1, 1), jnp.float32),
    )(partials)


def _relayout(flat):
    BB = 32
    ROWS = BB * T * VP // 128

    def body(i_ref, o_ref):
        x = i_ref[...].reshape(BB * T, VP)
        o_ref[...] = x[:, :V].reshape(BB, T, V)

    return pl.pallas_call(
        body,
        grid=(B // BB,),
        in_specs=[pl.BlockSpec((ROWS, 128), lambda i: (i, 0))],
        out_specs=pl.BlockSpec((BB, T, V), lambda i: (i, 0, 0)),
        out_shape=jax.ShapeDtypeStruct((B, T, V), jnp.float32),
    )(flat)


def _sc_gather_and_nll(table, idx_flat, tgt_flat, lse):
    mesh = plsc.VectorSubcoreMesh(core_axis_name="c", subcore_axis_name="s")
    cp = pltpu.CompilerParams()
    if "needs_layout_passes" in pltpu.CompilerParams.__dataclass_fields__:
        cp = dataclasses.replace(cp, needs_layout_passes=False)
    if "use_tc_tiling_on_sc" in pltpu.CompilerParams.__dataclass_fields__:
        cp = dataclasses.replace(cp, use_tc_tiling_on_sc=False)

    @functools.partial(
        pl.kernel,
        compiler_params=cp,
        out_type=(
            jax.ShapeDtypeStruct((N, VP), jnp.float32),
            jax.ShapeDtypeStruct((NW, L), jnp.float32),
        ),
        mesh=mesh,
        scratch_types=[
            pltpu.VMEM((CPT,), jnp.int32),
            pltpu.VMEM((CPT,), jnp.int32),
            pltpu.VMEM((V,), jnp.float32),
            pltpu.VMEM((RING, VP), jnp.float32),
            pltpu.VMEM((L,), jnp.float32),
            pltpu.SemaphoreType.DMA,
            [pltpu.SemaphoreType.DMA] * GS,
            [pltpu.SemaphoreType.DMA] * OS,
        ],
    )
    def k(table_hbm, idx_hbm, tgt_hbm, lse_hbm, out_hbm, part_hbm,
          idx_v, tgt_v, lse_v, ring_v, acc_v, ssem, gsems, osems):
        wid = lax.axis_index("s") * 2 + lax.axis_index("c")
        rbase = wid * CPT
        bbase = wid * BPT

        pltpu.async_copy(idx_hbm.at[pl.ds(rbase, CPT)], idx_v, ssem).wait()
        pltpu.async_copy(tgt_hbm.at[pl.ds(rbase, CPT)], tgt_v, ssem).wait()
        pltpu.async_copy(lse_hbm, lse_v, ssem).wait()
        acc_v[...] = jnp.zeros((L,), jnp.float32)

        def start_gather(c, u):
            pltpu.async_copy(
                table_hbm.at[idx_v.at[pl.ds(c * K, K)]],
                ring_v.at[pl.ds(u * K, K)], gsems[u])

        def wait_gather(c, u):
            pltpu.make_async_copy(
                table_hbm.at[idx_v.at[pl.ds(c * K, K)]],
                ring_v.at[pl.ds(u * K, K)], gsems[u]).wait()

        def start_out(j, v):
            pltpu.async_copy(
                ring_v.at[pl.ds(v * T, T)],
                out_hbm.at[pl.ds(rbase + j * T, T)], osems[v])

        def wait_out(j, v):
            pltpu.make_async_copy(
                ring_v.at[pl.ds(v * T, T)],
                out_hbm.at[pl.ds(rbase + j * T, T)], osems[v]).wait()

        start_gather(0, 0)
        start_gather(1, 1)

        iota = lax.iota(jnp.int32, L)

        @pl.loop(0, NC // GS)
        def _(r):
            c0 = r * GS
            j0 = r * OS
            for u in range(GS):
                c = c0 + u
                wait_gather(c, u)

                idx16 = idx_v[pl.ds(c * K, L)]
                tgt16 = tgt_v[pl.ds(c * K, L)]
                row16 = iota + jnp.int32(u * K)
                vals = plsc.load_gather(ring_v, [row16, tgt16])
                lsev = plsc.load_gather(lse_v, [idx16])
                acc_v[...] += lsev - vals

                if u >= 1:
                    start_out(j0 + u - 1, u - 1)

                u2 = (u + 2) % GS
                c2 = c + 2
                if u2 != GS - 1:
                    @pl.when(jnp.logical_and(c2 >= GS, c2 < NC))
                    def _():
                        wait_out(j0 + (OS if u + 2 >= GS else 0) + u2 - OS,
                                 u2)

                @pl.when(c2 < NC)
                def _():
                    start_gather(c2, u2)

        last_j0 = (NC // GS - 1) * OS
        for v in range(OS):
            wait_out(last_j0 + v, v)

        pltpu.sync_copy(acc_v, part_hbm.at[wid])

    return k(table, idx_flat, tgt_flat, lse)


def kernel(index, target, table):
    idx_flat = index.reshape(N).astype(jnp.int32)
    tgt_flat = target.reshape(N).astype(jnp.int32)
    lse = _lse_rows(table).reshape(V)
    table_pad = jnp.pad(table, ((0, 0), (0, VP - V)))
    flat, partials = _sc_gather_and_nll(table_pad, idx_flat, tgt_flat, lse)
    logits = _relayout(flat.reshape(N * VP // 128, 128))
    loss = _mean_partials(partials)[0, 0]
    return logits, loss


# speedup vs baseline: 1.0746x; 1.0746x over previous
import dataclasses
import functools

import jax
import jax.numpy as jnp
from jax import lax
from jax.experimental import pallas as pl
from jax.experimental.pallas import tpu as pltpu
from jax.experimental.pallas import tpu_sc as plsc

V = 1000
VP = 1024
B = 4096
T = 20
N = B * T
L = 16
NW = 32
BPT = B // NW
CPT = N // NW
K = 16
NC = CPT // K
RING = 80
GS = RING // K
OS = RING // T


def _lse_rows(table):
    def body(t_ref, o_ref):
        t = t_ref[...]
        m = jnp.max(t, axis=1, keepdims=True)
        s = jnp.sum(jnp.exp(t - m), axis=1, keepdims=True)
        o_ref[...] = jnp.log(s) + m

    return pl.pallas_call(
        body,
        out_shape=jax.ShapeDtypeStruct((V, 1), jnp.float32),
    )(table)


def _mean_partials(partials):
    def body(p_ref, o_ref):
        s = jnp.sum(p_ref[...], axis=1, keepdims=True)
        o_ref[...] = jnp.sum(s, axis=0, keepdims=True) * jnp.float32(1.0 / N)

    return pl.pallas_call(
        body,
        out_shape=jax.ShapeDtypeStruct((1, 1), jnp.float32),
    )(partials)


def _relayout(flat3):
    BB = 128

    def body(i_ref, o_ref):
        x = i_ref[...].reshape(BB, VP)
        o_ref[...] = jnp.transpose(x[:, :V], (1, 0))

    return pl.pallas_call(
        body,
        grid=(T, B // BB),
        in_specs=[pl.BlockSpec((BB, 8, 128), lambda t, b: (b, t, 0))],
        out_specs=pl.BlockSpec((V, BB), lambda t, b: (t, b)),
        out_shape=jax.ShapeDtypeStruct((T * V, B), jnp.float32),
    )(flat3)


def _sc_gather_and_nll(table, idx_flat, tgt_flat, lse):
    mesh = plsc.VectorSubcoreMesh(core_axis_name="c", subcore_axis_name="s")
    cp = pltpu.CompilerParams()
    if "needs_layout_passes" in pltpu.CompilerParams.__dataclass_fields__:
        cp = dataclasses.replace(cp, needs_layout_passes=False)
    if "use_tc_tiling_on_sc" in pltpu.CompilerParams.__dataclass_fields__:
        cp = dataclasses.replace(cp, use_tc_tiling_on_sc=False)

    @functools.partial(
        pl.kernel,
        compiler_params=cp,
        out_type=(
            jax.ShapeDtypeStruct((N, VP), jnp.float32),
            jax.ShapeDtypeStruct((NW, L), jnp.float32),
        ),
        mesh=mesh,
        scratch_types=[
            pltpu.VMEM((CPT,), jnp.int32),
            pltpu.VMEM((CPT,), jnp.int32),
            pltpu.VMEM((V,), jnp.float32),
            pltpu.VMEM((RING, VP), jnp.float32),
            pltpu.VMEM((L,), jnp.float32),
            pltpu.SemaphoreType.DMA,
            [pltpu.SemaphoreType.DMA] * GS,
            [pltpu.SemaphoreType.DMA] * OS,
        ],
    )
    def k(table_hbm, idx_hbm, tgt_hbm, lse_hbm, out_hbm, part_hbm,
          idx_v, tgt_v, lse_v, ring_v, acc_v, ssem, gsems, osems):
        wid = lax.axis_index("s") * 2 + lax.axis_index("c")
        rbase = wid * CPT
        bbase = wid * BPT

        pltpu.async_copy(idx_hbm.at[pl.ds(rbase, CPT)], idx_v, ssem).wait()
        pltpu.async_copy(tgt_hbm.at[pl.ds(rbase, CPT)], tgt_v, ssem).wait()
        pltpu.async_copy(lse_hbm, lse_v, ssem).wait()
        acc_v[...] = jnp.zeros((L,), jnp.float32)

        def start_gather(c, u):
            pltpu.async_copy(
                table_hbm.at[idx_v.at[pl.ds(c * K, K)]],
                ring_v.at[pl.ds(u * K, K)], gsems[u])

        def wait_gather(c, u):
            pltpu.make_async_copy(
                table_hbm.at[idx_v.at[pl.ds(c * K, K)]],
                ring_v.at[pl.ds(u * K, K)], gsems[u]).wait()

        def start_out(j, v):
            pltpu.async_copy(
                ring_v.at[pl.ds(v * T, T)],
                out_hbm.at[pl.ds(rbase + j * T, T)], osems[v])

        def wait_out(j, v):
            pltpu.make_async_copy(
                ring_v.at[pl.ds(v * T, T)],
                out_hbm.at[pl.ds(rbase + j * T, T)], osems[v]).wait()

        start_gather(0, 0)
        start_gather(1, 1)

        iota = lax.iota(jnp.int32, L)

        @pl.loop(0, NC // GS)
        def _(r):
            c0 = r * GS
            j0 = r * OS
            for u in range(GS):
                c = c0 + u
                wait_gather(c, u)

                idx16 = idx_v[pl.ds(c * K, L)]
                tgt16 = tgt_v[pl.ds(c * K, L)]
                row16 = iota + jnp.int32(u * K)
                vals = plsc.load_gather(ring_v, [row16, tgt16])
                lsev = plsc.load_gather(lse_v, [idx16])
                acc_v[...] += lsev - vals

                if u >= 1:
                    start_out(j0 + u - 1, u - 1)

                u2 = (u + 2) % GS
                c2 = c + 2
                if u2 != GS - 1:
                    @pl.when(jnp.logical_and(c2 >= GS, c2 < NC))
                    def _():
                        wait_out(j0 + (OS if u + 2 >= GS else 0) + u2 - OS,
                                 u2)

                @pl.when(c2 < NC)
                def _():
                    start_gather(c2, u2)

        last_j0 = (NC // GS - 1) * OS
        for v in range(OS):
            wait_out(last_j0 + v, v)

        pltpu.sync_copy(acc_v, part_hbm.at[wid])

    return k(table, idx_flat, tgt_flat, lse)


def kernel(index, target, table):
    idx_flat = index.reshape(N).astype(jnp.int32)
    tgt_flat = target.reshape(N).astype(jnp.int32)
    lse = _lse_rows(table).reshape(V)
    table_pad = jnp.pad(table, ((0, 0), (0, VP - V)))
    flat, partials = _sc_gather_and_nll(table_pad, idx_flat, tgt_flat, lse)
    logits_t = _relayout(flat.reshape(B, T * 8, 128))
    logits = jnp.transpose(logits_t.reshape(T, V, B), (2, 0, 1))
    loss = _mean_partials(partials)[0, 0]
    return logits, loss
